# Initial kernel scaffold; baseline (speedup 1.0000x reference)
import functools

import jax
import jax.numpy as jnp
from jax import lax
from jax.experimental import pallas as pl
from jax.experimental.pallas import tpu as pltpu
from jax.experimental.pallas import tpu_sc as plsc

NC = 2
NS = 16


def _support_body(h_ref, w_ref, b_ref, out_ref):
    out_ref[0] = (
        jnp.dot(h_ref[...], w_ref[0], preferred_element_type=jnp.float32)
        + b_ref[0, 0][None, :]
    )


def _tc_support(h, Wr, br3):
    N, H = h.shape
    R = Wr.shape[0]
    BM = 2000
    grid = (R, N // BM)
    return pl.pallas_call(
        _support_body,
        grid=grid,
        in_specs=[
            pl.BlockSpec((BM, H), lambda r, i: (i, 0)),
            pl.BlockSpec((1, H, H), lambda r, i: (r, 0, 0)),
            pl.BlockSpec((1, 1, H), lambda r, i: (r, 0, 0)),
        ],
        out_specs=pl.BlockSpec((1, BM, H), lambda r, i: (r, i, 0)),
        out_shape=jax.ShapeDtypeStruct((R, N, H), jnp.float32),
    )(h, Wr, br3)


def _sc_aggregate(sup_flat, src2d, dst2d, zrows, *, N, H, n_chunks, C):
    mesh = plsc.VectorSubcoreMesh(core_axis_name="c", subcore_axis_name="s")
    stripe = N // NS

    @functools.partial(
        pl.kernel,
        out_type=jax.ShapeDtypeStruct((NC, N, H), jnp.float32),
        mesh=mesh,
        scratch_types=[
            pltpu.VMEM((n_chunks, C), jnp.int32),
            pltpu.VMEM((n_chunks, C), jnp.int32),
            pltpu.VMEM((C, H), jnp.float32),
            pltpu.VMEM_SHARED((N, H), jnp.float32),
        ],
    )
    def agg_kernel(sup_hbm, src_hbm, dst_hbm, z_hbm, out_hbm,
                   src_v, dst_v, rows_v, agg_sh):
        cid = lax.axis_index("c")
        sid = lax.axis_index("s")
        wid = cid * NS + sid

        pltpu.sync_copy(z_hbm.at[pl.ds(sid * stripe, stripe)],
                        agg_sh.at[pl.ds(sid * stripe, stripe)])
        pltpu.sync_copy(src_hbm.at[pl.ds(wid * n_chunks, n_chunks)], src_v)
        pltpu.sync_copy(dst_hbm.at[pl.ds(wid * n_chunks, n_chunks)], dst_v)
        plsc.subcore_barrier()

        @pl.loop(0, n_chunks)
        def _(j):
            pltpu.sync_copy(sup_hbm.at[src_v.at[j]], rows_v)
            pltpu.sync_copy(rows_v, agg_sh.at[dst_v.at[j]], add=True)

        plsc.subcore_barrier()
        pltpu.sync_copy(agg_sh.at[pl.ds(sid * stripe, stripe)],
                        out_hbm.at[cid, pl.ds(sid * stripe, stripe)])

    return agg_kernel(sup_flat, src2d, dst2d, zrows)


def _gates_body(agg_ref, x_ref, h_ref, c_ref, ht_ref,
                wh_ref, wn_ref, wt_ref, u_ref, g_ref, v_ref, bv_ref,
                oh_ref, oc_ref):
    H = x_ref.shape[1]
    hn = jnp.tanh(agg_ref[0] + agg_ref[1])
    gv = jnp.dot(g_ref[...], v_ref[...], preferred_element_type=jnp.float32)
    gates = (
        jnp.dot(h_ref[...], wh_ref[...], preferred_element_type=jnp.float32)
        + jnp.dot(x_ref[...], u_ref[...], preferred_element_type=jnp.float32)
        + jnp.dot(hn, wn_ref[...], preferred_element_type=jnp.float32)
        + jnp.dot(ht_ref[...], wt_ref[...], preferred_element_type=jnp.float32)
        + gv + bv_ref[...]
    )
    i_g = gates[:, 0 * H:1 * H]
    f_g = gates[:, 1 * H:2 * H]
    o_g = gates[:, 2 * H:3 * H]
    u_g = gates[:, 3 * H:4 * H]
    t_g = gates[:, 4 * H:5 * H]
    new_c = (jax.nn.sigmoid(f_g) * c_ref[...]
             + jax.nn.sigmoid(i_g) * jnp.tanh(u_g)
             + jax.nn.sigmoid(t_g) * ht_ref[...])
    oc_ref[...] = new_c
    oh_ref[...] = jax.nn.sigmoid(o_g) * jnp.tanh(new_c)


def _tc_gates(agg2, x, h, c, h_t, Wh, Wn, Wt, U, g2, V, bV2):
    N, H = x.shape
    BM = 2000
    grid = (N // BM,)
    row_spec = pl.BlockSpec((BM, H), lambda i: (i, 0))
    w_spec = pl.BlockSpec((H, 5 * H), lambda i: (0, 0))
    return pl.pallas_call(
        _gates_body,
        grid=grid,
        in_specs=[
            pl.BlockSpec((NC, BM, H), lambda i: (0, i, 0)),
            row_spec, row_spec, row_spec, row_spec,
            w_spec, w_spec, w_spec, w_spec,
            pl.BlockSpec((1, H), lambda i: (0, 0)),
            w_spec,
            pl.BlockSpec((1, 5 * H), lambda i: (0, 0)),
        ],
        out_specs=[row_spec, row_spec],
        out_shape=[
            jax.ShapeDtypeStruct((N, H), jnp.float32),
            jax.ShapeDtypeStruct((N, H), jnp.float32),
        ],
    )(agg2, x, h, c, h_t, Wh, Wn, Wt, U, g2, V, bV2)


def kernel(x, h, c, g, h_t, edge_index, Wr, br, Wh, Wn, Wt, U, V, bV):
    N, H = h.shape
    R = Wr.shape[0]
    E = edge_index.shape[2]
    EE = R * E

    C = 120
    per_tile = EE // (NC * NS)
    n_chunks = per_tile // C
    assert per_tile % C == 0 and N % NS == 0

    support = _tc_support(h, Wr, br.reshape(R, 1, H))
    sup_flat = support.reshape(R * N, H)

    offs = (jnp.arange(R, dtype=edge_index.dtype) * N)[:, None]
    src2d = (edge_index[:, 0, :] + offs).reshape(EE // C, C)
    dst2d = edge_index[:, 1, :].reshape(EE // C, C)
    zrows = jnp.zeros((N, H), jnp.float32)

    agg2 = _sc_aggregate(sup_flat, src2d, dst2d, zrows,
                         N=N, H=H, n_chunks=n_chunks, C=C)

    return _tc_gates(agg2, x, h, c, h_t, Wh, Wn, Wt, U,
                     g.reshape(1, H), V, bV.reshape(1, 5 * H))


# speedup vs baseline: 6.6738x; 6.6738x over previous
import functools

import jax
import jax.numpy as jnp
from jax import lax
from jax.experimental import pallas as pl
from jax.experimental.pallas import tpu as pltpu
from jax.experimental.pallas import tpu_sc as plsc

NC = 2
NS = 16


def _support_body(h_ref, w_ref, b_ref, out_ref):
    out_ref[0] = (
        jnp.dot(h_ref[...], w_ref[0], preferred_element_type=jnp.float32)
        + b_ref[0, 0][None, :]
    )


def _tc_support(h, Wr, br3):
    N, H = h.shape
    R = Wr.shape[0]
    BM = 2000
    grid = (R, N // BM)
    return pl.pallas_call(
        _support_body,
        grid=grid,
        in_specs=[
            pl.BlockSpec((BM, H), lambda r, i: (i, 0)),
            pl.BlockSpec((1, H, H), lambda r, i: (r, 0, 0)),
            pl.BlockSpec((1, 1, H), lambda r, i: (r, 0, 0)),
        ],
        out_specs=pl.BlockSpec((1, BM, H), lambda r, i: (r, i, 0)),
        out_shape=jax.ShapeDtypeStruct((R, N, H), jnp.float32),
    )(h, Wr, br3)


def _sc_aggregate(sup_flat, src2d, dst2d, zrows, *, N_pad, H, n_chunks, C):
    mesh = plsc.VectorSubcoreMesh(core_axis_name="c", subcore_axis_name="s",
                                  num_cores=NC, num_subcores=NS)
    stripe = N_pad // NS

    KS = 16
    assert n_chunks % KS == 0

    @functools.partial(
        pl.kernel,
        out_type=jax.ShapeDtypeStruct((NC, N_pad, H), jnp.float32),
        mesh=mesh,
        scratch_types=[
            pltpu.VMEM((KS, C), jnp.int32),
            pltpu.VMEM((KS, C), jnp.int32),
            pltpu.VMEM((C, H), jnp.float32),
            pltpu.VMEM_SHARED((N_pad, H), jnp.float32),
        ],
    )
    def agg_kernel(sup_hbm, src_hbm, dst_hbm, z_hbm, out_hbm,
                   src_v, dst_v, rows_v, agg_sh):
        cid = lax.axis_index("c")
        sid = lax.axis_index("s")
        wid = cid * NS + sid

        pltpu.sync_copy(z_hbm.at[pl.ds(sid * stripe, stripe)],
                        agg_sh.at[pl.ds(sid * stripe, stripe)])
        plsc.subcore_barrier()

        @pl.loop(0, n_chunks // KS)
        def _(s):
            pltpu.sync_copy(src_hbm.at[pl.ds(wid * n_chunks + s * KS, KS)],
                            src_v)
            pltpu.sync_copy(dst_hbm.at[pl.ds(wid * n_chunks + s * KS, KS)],
                            dst_v)

            @pl.loop(0, KS)
            def _(j):
                pltpu.sync_copy(sup_hbm.at[src_v.at[j]], rows_v)
                pltpu.sync_copy(rows_v, agg_sh.at[dst_v.at[j]], add=True)

        plsc.subcore_barrier()
        pltpu.sync_copy(agg_sh.at[pl.ds(sid * stripe, stripe)],
                        out_hbm.at[cid, pl.ds(sid * stripe, stripe)])

    return agg_kernel(sup_flat, src2d, dst2d, zrows)


def _gates_body(agg_ref, x_ref, h_ref, c_ref, ht_ref,
                wh_ref, wn_ref, wt_ref, u_ref, g_ref, v_ref, bv_ref,
                oh_ref, oc_ref):
    H = x_ref.shape[1]
    hn = jnp.tanh(agg_ref[0] + agg_ref[1])
    gv = jnp.dot(g_ref[...], v_ref[...], preferred_element_type=jnp.float32)
    gates = (
        jnp.dot(h_ref[...], wh_ref[...], preferred_element_type=jnp.float32)
        + jnp.dot(x_ref[...], u_ref[...], preferred_element_type=jnp.float32)
        + jnp.dot(hn, wn_ref[...], preferred_element_type=jnp.float32)
        + jnp.dot(ht_ref[...], wt_ref[...], preferred_element_type=jnp.float32)
        + gv + bv_ref[...]
    )
    i_g = gates[:, 0 * H:1 * H]
    f_g = gates[:, 1 * H:2 * H]
    o_g = gates[:, 2 * H:3 * H]
    u_g = gates[:, 3 * H:4 * H]
    t_g = gates[:, 4 * H:5 * H]
    new_c = (jax.nn.sigmoid(f_g) * c_ref[...]
             + jax.nn.sigmoid(i_g) * jnp.tanh(u_g)
             + jax.nn.sigmoid(t_g) * ht_ref[...])
    oc_ref[...] = new_c
    oh_ref[...] = jax.nn.sigmoid(o_g) * jnp.tanh(new_c)


def _tc_gates(agg2, x, h, c, h_t, Wh, Wn, Wt, U, g2, V, bV2):
    N, H = x.shape
    BM = 2000
    grid = (N // BM,)
    row_spec = pl.BlockSpec((BM, H), lambda i: (i, 0))
    w_spec = pl.BlockSpec((H, 5 * H), lambda i: (0, 0))
    return pl.pallas_call(
        _gates_body,
        grid=grid,
        in_specs=[
            pl.BlockSpec((NC, BM, H), lambda i: (0, i, 0)),
            row_spec, row_spec, row_spec, row_spec,
            w_spec, w_spec, w_spec, w_spec,
            pl.BlockSpec((1, H), lambda i: (0, 0)),
            w_spec,
            pl.BlockSpec((1, 5 * H), lambda i: (0, 0)),
        ],
        out_specs=[row_spec, row_spec],
        out_shape=[
            jax.ShapeDtypeStruct((N, H), jnp.float32),
            jax.ShapeDtypeStruct((N, H), jnp.float32),
        ],
    )(agg2, x, h, c, h_t, Wh, Wn, Wt, U, g2, V, bV2)


def kernel(x, h, c, g, h_t, edge_index, Wr, br, Wh, Wn, Wt, U, V, bV):
    N, H = h.shape
    R = Wr.shape[0]
    E = edge_index.shape[2]
    EE = R * E
    NT = NC * NS

    C = 128
    n_chunks = -(-EE // (NT * C * 8)) * 8
    EEp = NT * C * n_chunks
    pad = EEp - EE
    N_pad = -(-(N + (8 if pad else 0)) // (8 * NS)) * (8 * NS)
    n_dummy = N_pad - N

    support = _tc_support(h, Wr, br.reshape(R, 1, H))
    sup_flat = support.reshape(R * N, H)

    offs = (jnp.arange(R, dtype=edge_index.dtype) * N)[:, None]
    srcp = (edge_index[:, 0, :] + offs).reshape(EE)
    dstp = edge_index[:, 1, :].reshape(EE)
    if pad:
        fill = jnp.arange(pad, dtype=srcp.dtype)
        srcp = jnp.concatenate([srcp, fill % (R * N)])
        dstp = jnp.concatenate([dstp, N + fill % n_dummy])
    src2d = srcp.reshape(n_chunks, NT, C).transpose(1, 0, 2).reshape(EEp // C, C)
    dst2d = dstp.reshape(n_chunks, NT, C).transpose(1, 0, 2).reshape(EEp // C, C)
    zrows = jnp.zeros((N_pad, H), jnp.float32)

    agg2 = _sc_aggregate(sup_flat, src2d, dst2d, zrows,
                         N_pad=N_pad, H=H, n_chunks=n_chunks, C=C)

    return _tc_gates(agg2, x, h, c, h_t, Wh, Wn, Wt, U,
                     g.reshape(1, H), V, bV.reshape(1, 5 * H))


# speedup vs baseline: 9.8927x; 1.4823x over previous
import functools

import jax
import jax.numpy as jnp
from jax import lax
from jax.experimental import pallas as pl
from jax.experimental.pallas import tpu as pltpu
from jax.experimental.pallas import tpu_sc as plsc

NC = 2
NS = 16


def _support_body(h_ref, w_ref, b_ref, out_ref):
    out_ref[0] = (
        jnp.dot(h_ref[...], w_ref[0], preferred_element_type=jnp.float32)
        + b_ref[0, 0][None, :]
    )


def _tc_support(h, Wr, br3):
    N, H = h.shape
    R = Wr.shape[0]
    BM = 2000
    grid = (R, N // BM)
    return pl.pallas_call(
        _support_body,
        grid=grid,
        in_specs=[
            pl.BlockSpec((BM, H), lambda r, i: (i, 0)),
            pl.BlockSpec((1, H, H), lambda r, i: (r, 0, 0)),
            pl.BlockSpec((1, 1, H), lambda r, i: (r, 0, 0)),
        ],
        out_specs=pl.BlockSpec((1, BM, H), lambda r, i: (r, i, 0)),
        out_shape=jax.ShapeDtypeStruct((R, N, H), jnp.float32),
    )(h, Wr, br3)


def _sc_aggregate(sup_flat, src2d, dst2d, zrows, *, N_pad, H, n_chunks, C):
    mesh = plsc.VectorSubcoreMesh(core_axis_name="c", subcore_axis_name="s",
                                  num_cores=NC, num_subcores=NS)
    stripe = N_pad // NS

    KS = 24
    n_super = n_chunks // KS
    assert n_chunks % KS == 0 and n_super % 2 == 0 and KS % 2 == 0

    @functools.partial(
        pl.kernel,
        out_type=jax.ShapeDtypeStruct((NC, N_pad, H), jnp.float32),
        mesh=mesh,
        scratch_types=[
            pltpu.VMEM((KS, C), jnp.int32),
            pltpu.VMEM((KS, C), jnp.int32),
            pltpu.VMEM((KS, C), jnp.int32),
            pltpu.VMEM((KS, C), jnp.int32),
            pltpu.VMEM((C, H), jnp.float32),
            pltpu.VMEM((C, H), jnp.float32),
            pltpu.VMEM_SHARED((N_pad, H), jnp.float32),
            pltpu.SemaphoreType.DMA,
            pltpu.SemaphoreType.DMA,
            pltpu.SemaphoreType.DMA,
            pltpu.SemaphoreType.DMA,
        ],
    )
    def agg_kernel(sup_hbm, src_hbm, dst_hbm, z_hbm, out_hbm,
                   srcA, dstA, srcB, dstB, rowsA, rowsB, agg_sh,
                   isemA, isemB, gsA, gsB):
        cid = lax.axis_index("c")
        sid = lax.axis_index("s")
        wid = cid * NS + sid
        base = wid * n_chunks

        def drain_idx(dst_ref, sem):
            pltpu.make_async_copy(src_hbm.at[pl.ds(0, KS)], dst_ref, sem).wait()

        def drain_rows(rows_ref, sem):
            pltpu.make_async_copy(sup_hbm.at[srcA.at[0]], rows_ref, sem).wait()

        def super_body(src_v, dst_v):
            pltpu.async_copy(sup_hbm.at[src_v.at[0]], rowsA, gsA)

            @pl.loop(0, KS // 2 - 1)
            def _(t):
                j = 2 * t
                pltpu.async_copy(sup_hbm.at[src_v.at[j + 1]], rowsB, gsB)
                drain_rows(rowsA, gsA)
                pltpu.sync_copy(rowsA, agg_sh.at[dst_v.at[j]], add=True)
                pltpu.async_copy(sup_hbm.at[src_v.at[j + 2]], rowsA, gsA)
                drain_rows(rowsB, gsB)
                pltpu.sync_copy(rowsB, agg_sh.at[dst_v.at[j + 1]], add=True)

            pltpu.async_copy(sup_hbm.at[src_v.at[KS - 1]], rowsB, gsB)
            drain_rows(rowsA, gsA)
            pltpu.sync_copy(rowsA, agg_sh.at[dst_v.at[KS - 2]], add=True)
            drain_rows(rowsB, gsB)
            pltpu.sync_copy(rowsB, agg_sh.at[dst_v.at[KS - 1]], add=True)

        pltpu.sync_copy(z_hbm.at[pl.ds(sid * stripe, stripe)],
                        agg_sh.at[pl.ds(sid * stripe, stripe)])
        pltpu.async_copy(src_hbm.at[pl.ds(base, KS)], srcA, isemA)
        pltpu.async_copy(dst_hbm.at[pl.ds(base, KS)], dstA, isemA)
        plsc.subcore_barrier()

        @pl.loop(0, n_super // 2)
        def _(ss):
            s0 = 2 * ss
            pltpu.async_copy(src_hbm.at[pl.ds(base + (s0 + 1) * KS, KS)],
                             srcB, isemB)
            pltpu.async_copy(dst_hbm.at[pl.ds(base + (s0 + 1) * KS, KS)],
                             dstB, isemB)
            drain_idx(srcA, isemA)
            drain_idx(dstA, isemA)
            super_body(srcA, dstA)

            @pl.when(ss < n_super // 2 - 1)
            def _():
                pltpu.async_copy(src_hbm.at[pl.ds(base + (s0 + 2) * KS, KS)],
                                 srcA, isemA)
                pltpu.async_copy(dst_hbm.at[pl.ds(base + (s0 + 2) * KS, KS)],
                                 dstA, isemA)

            drain_idx(srcB, isemB)
            drain_idx(dstB, isemB)
            super_body(srcB, dstB)

        plsc.subcore_barrier()
        pltpu.sync_copy(agg_sh.at[pl.ds(sid * stripe, stripe)],
                        out_hbm.at[cid, pl.ds(sid * stripe, stripe)])

    return agg_kernel(sup_flat, src2d, dst2d, zrows)


def _gates_body(agg_ref, x_ref, h_ref, c_ref, ht_ref,
                wh_ref, wn_ref, wt_ref, u_ref, g_ref, v_ref, bv_ref,
                oh_ref, oc_ref):
    H = x_ref.shape[1]
    hn = jnp.tanh(agg_ref[0] + agg_ref[1])
    gv = jnp.dot(g_ref[...], v_ref[...], preferred_element_type=jnp.float32)
    gates = (
        jnp.dot(h_ref[...], wh_ref[...], preferred_element_type=jnp.float32)
        + jnp.dot(x_ref[...], u_ref[...], preferred_element_type=jnp.float32)
        + jnp.dot(hn, wn_ref[...], preferred_element_type=jnp.float32)
        + jnp.dot(ht_ref[...], wt_ref[...], preferred_element_type=jnp.float32)
        + gv + bv_ref[...]
    )
    i_g = gates[:, 0 * H:1 * H]
    f_g = gates[:, 1 * H:2 * H]
    o_g = gates[:, 2 * H:3 * H]
    u_g = gates[:, 3 * H:4 * H]
    t_g = gates[:, 4 * H:5 * H]
    new_c = (jax.nn.sigmoid(f_g) * c_ref[...]
             + jax.nn.sigmoid(i_g) * jnp.tanh(u_g)
             + jax.nn.sigmoid(t_g) * ht_ref[...])
    oc_ref[...] = new_c
    oh_ref[...] = jax.nn.sigmoid(o_g) * jnp.tanh(new_c)


def _tc_gates(agg2, x, h, c, h_t, Wh, Wn, Wt, U, g2, V, bV2):
    N, H = x.shape
    BM = 2000
    grid = (N // BM,)
    row_spec = pl.BlockSpec((BM, H), lambda i: (i, 0))
    w_spec = pl.BlockSpec((H, 5 * H), lambda i: (0, 0))
    return pl.pallas_call(
        _gates_body,
        grid=grid,
        in_specs=[
            pl.BlockSpec((NC, BM, H), lambda i: (0, i, 0)),
            row_spec, row_spec, row_spec, row_spec,
            w_spec, w_spec, w_spec, w_spec,
            pl.BlockSpec((1, H), lambda i: (0, 0)),
            w_spec,
            pl.BlockSpec((1, 5 * H), lambda i: (0, 0)),
        ],
        out_specs=[row_spec, row_spec],
        out_shape=[
            jax.ShapeDtypeStruct((N, H), jnp.float32),
            jax.ShapeDtypeStruct((N, H), jnp.float32),
        ],
    )(agg2, x, h, c, h_t, Wh, Wn, Wt, U, g2, V, bV2)


def kernel(x, h, c, g, h_t, edge_index, Wr, br, Wh, Wn, Wt, U, V, bV):
    N, H = h.shape
    R = Wr.shape[0]
    E = edge_index.shape[2]
    EE = R * E
    NT = NC * NS

    C = 128
    n_chunks = -(-EE // (NT * C * 8)) * 8
    EEp = NT * C * n_chunks
    pad = EEp - EE
    N_pad = -(-(N + (8 if pad else 0)) // (8 * NS)) * (8 * NS)
    n_dummy = N_pad - N

    support = _tc_support(h, Wr, br.reshape(R, 1, H))
    sup_flat = support.reshape(R * N, H)

    offs = (jnp.arange(R, dtype=edge_index.dtype) * N)[:, None]
    srcp = (edge_index[:, 0, :] + offs).reshape(EE)
    dstp = edge_index[:, 1, :].reshape(EE)
    if pad:
        fill = jnp.arange(pad, dtype=srcp.dtype)
        srcp = jnp.concatenate([srcp, fill % (R * N)])
        dstp = jnp.concatenate([dstp, N + fill % n_dummy])
    src2d = srcp.reshape(n_chunks, NT, C).transpose(1, 0, 2).reshape(EEp // C, C)
    dst2d = dstp.reshape(n_chunks, NT, C).transpose(1, 0, 2).reshape(EEp // C, C)
    zrows = jnp.zeros((N_pad, H), jnp.float32)

    agg2 = _sc_aggregate(sup_flat, src2d, dst2d, zrows,
                         N_pad=N_pad, H=H, n_chunks=n_chunks, C=C)

    return _tc_gates(agg2, x, h, c, h_t, Wh, Wn, Wt, U,
                     g.reshape(1, H), V, bV.reshape(1, 5 * H))
